# baseline (speedup 1.0000x reference)
import functools

import jax
import jax.numpy as jnp
from jax import lax
from jax.experimental import pallas as pl
from jax.experimental.pallas import tpu as pltpu
from jax.experimental.pallas import tpu_sc as plsc

ENTRIES = 1024
DIM = 64
BATCH = 16
TOK = 576
ROWS = BATCH * TOK
LANES = 128
SUB = 8
BPS = 16
_TILES = [(0, 128), (128, 128), (256, 128), (384, 128), (512, 64)]


def _argmin_body(a2_ref, b2t_ref, zt_ref, cb_ref, idx_ref):
    b2t = b2t_ref[...]
    for b in range(BPS):
        mmT = lax.dot_general(
            cb_ref[...], zt_ref[b],
            dimension_numbers=(((1,), (0,)), ((), ())),
            preferred_element_type=jnp.float32,
        )
        for clo, w in _TILES:
            a2c = a2_ref[b:b + 1, clo:clo + w]
            ii0 = lax.broadcasted_iota(jnp.int32, (SUB, w), 0)
            best_v = best_i = None
            for k in range(ENTRIES // SUB):
                bk = jnp.broadcast_to(b2t[:, k:k + 1], (SUB, w))
                d = (bk + a2c) - 2.0 * mmT[k * SUB:(k + 1) * SUB, clo:clo + w]
                if k == 0:
                    best_v, best_i = d, ii0
                else:
                    upd = d < best_v
                    best_v = jnp.minimum(best_v, d)
                    best_i = jnp.where(upd, ii0 + jnp.int32(k * SUB), best_i)
            m = jnp.min(best_v, axis=0, keepdims=True)
            idx_ref[b, clo:clo + w] = jnp.min(
                jnp.where(best_v == m, best_i, jnp.int32(ENTRIES)), axis=0)


def _nearest_idx(z, codebook):
    flat = z.reshape(-1, DIM)
    a2 = jnp.sum(flat ** 2, axis=1).reshape(BATCH, TOK)
    b2t = jnp.sum(codebook ** 2, axis=1).reshape(LANES, SUB).T
    zt = jnp.transpose(z, (0, 2, 1))
    return pl.pallas_call(
        _argmin_body,
        grid=(BATCH // BPS,),
        in_specs=[
            pl.BlockSpec((BPS, TOK), lambda i: (i, 0)),
            pl.BlockSpec((SUB, LANES), lambda i: (0, 0)),
            pl.BlockSpec((BPS, DIM, TOK), lambda i: (i, 0, 0)),
            pl.BlockSpec((ENTRIES, DIM), lambda i: (0, 0)),
        ],
        out_specs=pl.BlockSpec((BPS, TOK), lambda i: (i, 0)),
        out_shape=jax.ShapeDtypeStruct((BATCH, TOK), jnp.int32),
    )(a2, b2t, zt, codebook)


@functools.cache
def _make_gather():
    nc, ns = 2, 16
    nw = nc * ns
    b_per_w = ROWS // nw
    halves = nw // BATCH
    chunk = 96
    n_chunks = b_per_w // chunk
    mesh = plsc.VectorSubcoreMesh(core_axis_name="c", subcore_axis_name="s")

    @functools.partial(
        pl.kernel, mesh=mesh,
        compiler_params=pltpu.CompilerParams(use_tc_tiling_on_sc=False),
        out_type=jax.ShapeDtypeStruct((BATCH, TOK, DIM), jnp.float32),
        scratch_types=[
            pltpu.VMEM((b_per_w,), jnp.int32),
            pltpu.VMEM((b_per_w, DIM), jnp.float32),
            pltpu.SemaphoreType.DMA,
            pltpu.SemaphoreType.DMA,
        ],
    )
    def gather(table_hbm, idx_hbm, out_hbm, idx_v, rows_v, gsem, wsem):
        wid = lax.axis_index("s") * nc + lax.axis_index("c")
        b = wid // halves
        t0 = (wid % halves) * b_per_w
        pltpu.sync_copy(idx_hbm.at[b, pl.ds(t0, b_per_w)], idx_v)
        gathers = [
            pltpu.async_copy(
                table_hbm.at[idx_v.at[pl.ds(k * chunk, chunk)]],
                rows_v.at[pl.ds(k * chunk, chunk)],
                gsem,
            )
            for k in range(n_chunks)
        ]
        writes = []
        for k in range(n_chunks):
            gathers[k].wait()
            writes.append(pltpu.async_copy(
                rows_v.at[pl.ds(k * chunk, chunk)],
                out_hbm.at[b, pl.ds(t0 + k * chunk, chunk)],
                wsem,
            ))
        for w in writes:
            w.wait()

    return gather


def kernel(z, codebook):
    idx = _nearest_idx(z, codebook)
    z_q = _make_gather()(codebook, idx)
    return z_q, idx


# speedup vs baseline: 1.0044x; 1.0044x over previous
import functools

import jax
import jax.numpy as jnp
from jax import lax
from jax.experimental import pallas as pl
from jax.experimental.pallas import tpu as pltpu
from jax.experimental.pallas import tpu_sc as plsc

ENTRIES = 1024
DIM = 64
BATCH = 16
TOK = 576
ROWS = BATCH * TOK
LANES = 128
SUB = 8
BPS = 8
_TILES = [(0, 128), (128, 128), (256, 128), (384, 128), (512, 64)]


def _argmin_body(a2_ref, b2t_ref, zt_ref, cb_ref, idx_ref):
    b2t = b2t_ref[...]
    for b in range(BPS):
        mmT = lax.dot_general(
            cb_ref[...], zt_ref[b],
            dimension_numbers=(((1,), (0,)), ((), ())),
            preferred_element_type=jnp.float32,
        )
        for clo, w in _TILES:
            a2c = a2_ref[b:b + 1, clo:clo + w]
            ii0 = lax.broadcasted_iota(jnp.int32, (SUB, w), 0)
            best_v = best_i = None
            for k in range(ENTRIES // SUB):
                bk = jnp.broadcast_to(b2t[:, k:k + 1], (SUB, w))
                d = (bk + a2c) - 2.0 * mmT[k * SUB:(k + 1) * SUB, clo:clo + w]
                if k == 0:
                    best_v, best_i = d, ii0
                else:
                    upd = d < best_v
                    best_v = jnp.minimum(best_v, d)
                    best_i = jnp.where(upd, ii0 + jnp.int32(k * SUB), best_i)
            m = jnp.min(best_v, axis=0, keepdims=True)
            idx_ref[b, clo:clo + w] = jnp.min(
                jnp.where(best_v == m, best_i, jnp.int32(ENTRIES)), axis=0)


def _nearest_idx(z, codebook):
    flat = z.reshape(-1, DIM)
    a2 = jnp.sum(flat ** 2, axis=1).reshape(BATCH, TOK)
    b2t = jnp.sum(codebook ** 2, axis=1).reshape(LANES, SUB).T
    zt = jnp.transpose(z, (0, 2, 1))
    return pl.pallas_call(
        _argmin_body,
        grid=(BATCH // BPS,),
        in_specs=[
            pl.BlockSpec((BPS, TOK), lambda i: (i, 0)),
            pl.BlockSpec((SUB, LANES), lambda i: (0, 0)),
            pl.BlockSpec((BPS, DIM, TOK), lambda i: (i, 0, 0)),
            pl.BlockSpec((ENTRIES, DIM), lambda i: (0, 0)),
        ],
        out_specs=pl.BlockSpec((BPS, TOK), lambda i: (i, 0)),
        out_shape=jax.ShapeDtypeStruct((BATCH, TOK), jnp.int32),
    )(a2, b2t, zt, codebook)


@functools.cache
def _make_gather():
    nc, ns = 2, 16
    nw = nc * ns
    b_per_w = ROWS // nw
    halves = nw // BATCH
    chunk = 96
    n_chunks = b_per_w // chunk
    mesh = plsc.VectorSubcoreMesh(core_axis_name="c", subcore_axis_name="s")

    @functools.partial(
        pl.kernel, mesh=mesh,
        compiler_params=pltpu.CompilerParams(use_tc_tiling_on_sc=False),
        out_type=jax.ShapeDtypeStruct((BATCH, TOK, DIM), jnp.float32),
        scratch_types=[
            pltpu.VMEM((b_per_w,), jnp.int32),
            pltpu.VMEM((b_per_w, DIM), jnp.float32),
            pltpu.SemaphoreType.DMA,
            pltpu.SemaphoreType.DMA,
        ],
    )
    def gather(table_hbm, idx_hbm, out_hbm, idx_v, rows_v, gsem, wsem):
        wid = lax.axis_index("s") * nc + lax.axis_index("c")
        b = wid // halves
        t0 = (wid % halves) * b_per_w
        pltpu.sync_copy(idx_hbm.at[b, pl.ds(t0, b_per_w)], idx_v)
        gathers = [
            pltpu.async_copy(
                table_hbm.at[idx_v.at[pl.ds(k * chunk, chunk)]],
                rows_v.at[pl.ds(k * chunk, chunk)],
                gsem,
            )
            for k in range(n_chunks)
        ]
        writes = []
        for k in range(n_chunks):
            gathers[k].wait()
            writes.append(pltpu.async_copy(
                rows_v.at[pl.ds(k * chunk, chunk)],
                out_hbm.at[b, pl.ds(t0 + k * chunk, chunk)],
                wsem,
            ))
        for w in writes:
            w.wait()

    return gather


def kernel(z, codebook):
    idx = _nearest_idx(z, codebook)
    z_q = _make_gather()(codebook, idx)
    return z_q, idx
